# baseline (speedup 1.0000x reference)
import jax
import jax.numpy as jnp
from jax.experimental import pallas as pl

_N_ENT = 1_000_000
_D = 32
_BLK = 8000


def _huber_sum_block(e):
    ae = jnp.abs(e)
    return jnp.sum(jnp.where(ae < 1.0, 0.5 * e * e, ae - 0.5),
                   keepdims=True).reshape(1, 1)


def _body(ent_ref, old_ent_ref, mask_ref, rel_ref, old_rel_ref, out_ref):
    i = pl.program_id(0)

    @pl.when(i == 0)
    def _init():
        out_ref[...] = _huber_sum_block(rel_ref[...] - old_rel_ref[...])

    m = mask_ref[0, 0, :]
    e = (ent_ref[...] - old_ent_ref[...]) * m[:, None]
    out_ref[...] += _huber_sum_block(e)


def kernel(ent_embeddings, rel_embeddings, old_ent_embeddings,
           old_rel_embeddings, entity_distill_mask):
    nb = _N_ENT // _BLK
    mask3 = entity_distill_mask.reshape(nb, 1, _BLK)
    out = pl.pallas_call(
        _body,
        grid=(nb,),
        in_specs=[
            pl.BlockSpec((_BLK, _D), lambda i: (i, 0)),
            pl.BlockSpec((_BLK, _D), lambda i: (i, 0)),
            pl.BlockSpec((1, 1, _BLK), lambda i: (i, 0, 0)),
            pl.BlockSpec(rel_embeddings.shape, lambda i: (0, 0)),
            pl.BlockSpec(old_rel_embeddings.shape, lambda i: (0, 0)),
        ],
        out_specs=pl.BlockSpec((1, 1), lambda i: (0, 0)),
        out_shape=jax.ShapeDtypeStruct((1, 1), jnp.float32),
    )(ent_embeddings, old_ent_embeddings, mask3,
      rel_embeddings, old_rel_embeddings)
    return out[0, 0]
